# baseline (speedup 1.0000x reference)
import functools

import jax
import jax.numpy as jnp
from jax import lax
from jax.experimental import pallas as pl
from jax.experimental.pallas import tpu as pltpu
from jax.experimental.pallas import tpu_sc as plsc

EMBED_DIM = 128
NUM_TABLE_ROWS = 1024
NUM_CORES = 2
NUM_SUBCORES = 16
NUM_WORKERS = NUM_CORES * NUM_SUBCORES
CHUNK = 128
DEPTH = 4


def _make_lookup(batch):
    assert batch % (8 * NUM_WORKERS) == 0
    b_per_w = batch // NUM_WORKERS
    assert b_per_w % (DEPTH * CHUNK) == 0
    n_chunks = b_per_w // CHUNK
    steps = n_chunks // DEPTH
    mesh = plsc.VectorSubcoreMesh(core_axis_name="c", subcore_axis_name="s")

    @functools.partial(
        pl.kernel,
        mesh=mesh,
        out_type=jax.ShapeDtypeStruct((batch, EMBED_DIM), jnp.float32),
        scratch_types=(
            [pltpu.VMEM_SHARED((NUM_TABLE_ROWS, EMBED_DIM), jnp.float32)]
            + [pltpu.VMEM((CHUNK,), jnp.int32) for _ in range(DEPTH)]
            + [pltpu.VMEM((CHUNK, EMBED_DIM), jnp.float32) for _ in range(DEPTH)]
            + [pltpu.SemaphoreType.DMA for _ in range(3 * DEPTH)]
        ),
    )
    def lookup(table_hbm, idx_hbm, out_hbm, tab_sh, *bufs):
        idx = bufs[:DEPTH]
        rows = bufs[DEPTH:2 * DEPTH]
        isem = bufs[2 * DEPTH:3 * DEPTH]
        gsem = bufs[3 * DEPTH:4 * DEPTH]
        ssem = bufs[4 * DEPTH:5 * DEPTH]

        sid = lax.axis_index("s")
        wid = sid * NUM_CORES + lax.axis_index("c")
        base = wid * b_per_w

        def idx_src(c):
            return idx_hbm.at[pl.ds(base + c * CHUNK, CHUNK)]

        def out_dst(c):
            return out_hbm.at[pl.ds(base + c * CHUNK, CHUNK)]

        @pl.when(sid == 0)
        def _():
            pltpu.sync_copy(table_hbm, tab_sh)

        plsc.subcore_barrier()

        for b in range(DEPTH):
            pltpu.async_copy(idx_src(b), idx[b], isem[b])
        pltpu.make_async_copy(idx_src(0), idx[0], isem[0]).wait()
        pltpu.async_copy(tab_sh.at[idx[0]], rows[0], gsem[0])

        def body(g, carry):
            for b in range(DEPTH):
                c = DEPTH * g + b
                nb = (b + 1) % DEPTH
                pltpu.make_async_copy(tab_sh.at[idx[b]], rows[b],
                                      gsem[b]).wait()
                pltpu.async_copy(rows[b], out_dst(c), ssem[b])

                @pl.when(c + DEPTH < n_chunks)
                def _():
                    pltpu.async_copy(idx_src(c + DEPTH), idx[b], isem[b])

                @pl.when((c + 1 < n_chunks) & (c + 1 >= DEPTH))
                def _():
                    pltpu.make_async_copy(
                        rows[nb], out_dst(c + 1 - DEPTH), ssem[nb]).wait()

                @pl.when(c + 1 < n_chunks)
                def _():
                    pltpu.make_async_copy(idx_src(c + 1), idx[nb],
                                          isem[nb]).wait()
                    pltpu.async_copy(tab_sh.at[idx[nb]], rows[nb], gsem[nb])

            return carry

        lax.fori_loop(0, steps, body, 0)

        for b in range(DEPTH):
            c = n_chunks - DEPTH + b
            pltpu.make_async_copy(rows[c % DEPTH], out_dst(c),
                                  ssem[c % DEPTH]).wait()

    return lookup


def kernel(weights, positions):
    flat = positions.reshape(-1).astype(jnp.int32)
    out = _make_lookup(flat.shape[0])(weights, flat)
    return out.reshape(*positions.shape, EMBED_DIM)


# speedup vs baseline: 1.0316x; 1.0316x over previous
import functools

import jax
import jax.numpy as jnp
from jax import lax
from jax.experimental import pallas as pl
from jax.experimental.pallas import tpu as pltpu
from jax.experimental.pallas import tpu_sc as plsc

EMBED_DIM = 128
NUM_TABLE_ROWS = 1024
NUM_CORES = 2
NUM_SUBCORES = 16
NUM_WORKERS = NUM_CORES * NUM_SUBCORES
CHUNK = 200
DEPTH = 4


def _make_lookup(batch):
    assert batch % (8 * NUM_WORKERS) == 0
    b_per_w = batch // NUM_WORKERS
    assert b_per_w % (DEPTH * CHUNK) == 0
    n_chunks = b_per_w // CHUNK
    steps = n_chunks // DEPTH
    mesh = plsc.VectorSubcoreMesh(core_axis_name="c", subcore_axis_name="s")

    @functools.partial(
        pl.kernel,
        mesh=mesh,
        out_type=jax.ShapeDtypeStruct((batch, EMBED_DIM), jnp.float32),
        scratch_types=(
            [pltpu.VMEM_SHARED((NUM_TABLE_ROWS, EMBED_DIM), jnp.float32)]
            + [pltpu.VMEM((CHUNK,), jnp.int32) for _ in range(DEPTH)]
            + [pltpu.VMEM((CHUNK, EMBED_DIM), jnp.float32) for _ in range(DEPTH)]
            + [pltpu.SemaphoreType.DMA for _ in range(3 * DEPTH)]
        ),
    )
    def lookup(table_hbm, idx_hbm, out_hbm, tab_sh, *bufs):
        idx = bufs[:DEPTH]
        rows = bufs[DEPTH:2 * DEPTH]
        isem = bufs[2 * DEPTH:3 * DEPTH]
        gsem = bufs[3 * DEPTH:4 * DEPTH]
        ssem = bufs[4 * DEPTH:5 * DEPTH]

        sid = lax.axis_index("s")
        wid = sid * NUM_CORES + lax.axis_index("c")
        base = wid * b_per_w

        def idx_src(c):
            return idx_hbm.at[pl.ds(base + c * CHUNK, CHUNK)]

        def out_dst(c):
            return out_hbm.at[pl.ds(base + c * CHUNK, CHUNK)]

        @pl.when(sid == 0)
        def _():
            pltpu.sync_copy(table_hbm, tab_sh)

        plsc.subcore_barrier()

        for b in range(DEPTH):
            pltpu.async_copy(idx_src(b), idx[b], isem[b])
        pltpu.make_async_copy(idx_src(0), idx[0], isem[0]).wait()
        pltpu.async_copy(tab_sh.at[idx[0]], rows[0], gsem[0])

        def body(g, carry):
            for b in range(DEPTH):
                c = DEPTH * g + b
                nb = (b + 1) % DEPTH
                pltpu.make_async_copy(tab_sh.at[idx[b]], rows[b],
                                      gsem[b]).wait()
                pltpu.async_copy(rows[b], out_dst(c), ssem[b])

                @pl.when(c + DEPTH < n_chunks)
                def _():
                    pltpu.async_copy(idx_src(c + DEPTH), idx[b], isem[b])

                @pl.when((c + 1 < n_chunks) & (c + 1 >= DEPTH))
                def _():
                    pltpu.make_async_copy(
                        rows[nb], out_dst(c + 1 - DEPTH), ssem[nb]).wait()

                @pl.when(c + 1 < n_chunks)
                def _():
                    pltpu.make_async_copy(idx_src(c + 1), idx[nb],
                                          isem[nb]).wait()
                    pltpu.async_copy(tab_sh.at[idx[nb]], rows[nb], gsem[nb])

            return carry

        lax.fori_loop(0, steps, body, 0)

        for b in range(DEPTH):
            c = n_chunks - DEPTH + b
            pltpu.make_async_copy(rows[c % DEPTH], out_dst(c),
                                  ssem[c % DEPTH]).wait()

    return lookup


def kernel(weights, positions):
    flat = positions.reshape(-1).astype(jnp.int32)
    out = _make_lookup(flat.shape[0])(weights, flat)
    return out.reshape(*positions.shape, EMBED_DIM)


# speedup vs baseline: 1.0588x; 1.0264x over previous
import functools

import jax
import jax.numpy as jnp
from jax import lax
from jax.experimental import pallas as pl
from jax.experimental.pallas import tpu as pltpu
from jax.experimental.pallas import tpu_sc as plsc

EMBED_DIM = 128
NUM_TABLE_ROWS = 1024
NUM_CORES = 2
NUM_SUBCORES = 16
NUM_WORKERS = NUM_CORES * NUM_SUBCORES
CHUNK = 256


def _make_lookup(batch):
    assert batch % (8 * NUM_WORKERS) == 0
    b_per_w = batch // NUM_WORKERS
    assert b_per_w % (2 * CHUNK) == 0
    n_chunks = b_per_w // CHUNK
    pair_steps = n_chunks // 2
    mesh = plsc.VectorSubcoreMesh(core_axis_name="c", subcore_axis_name="s")

    @functools.partial(
        pl.kernel,
        mesh=mesh,
        out_type=jax.ShapeDtypeStruct((batch, EMBED_DIM), jnp.float32),
        scratch_types=[
            pltpu.VMEM_SHARED((NUM_TABLE_ROWS, EMBED_DIM), jnp.float32),
            pltpu.VMEM((CHUNK,), jnp.int32),
            pltpu.VMEM((CHUNK,), jnp.int32),
            pltpu.VMEM((CHUNK, EMBED_DIM), jnp.float32),
            pltpu.VMEM((CHUNK, EMBED_DIM), jnp.float32),
            pltpu.SemaphoreType.DMA,
            pltpu.SemaphoreType.DMA,
            pltpu.SemaphoreType.DMA,
            pltpu.SemaphoreType.DMA,
        ],
    )
    def lookup(table_hbm, idx_hbm, out_hbm, tab_sh, idx0, idx1, rows0, rows1,
               gsem0, gsem1, isem0, isem1):
        sid = lax.axis_index("s")
        wid = sid * NUM_CORES + lax.axis_index("c")
        base = wid * b_per_w

        def idx_src(c):
            return idx_hbm.at[pl.ds(base + c * CHUNK, CHUNK)]

        def out_dst(c):
            return out_hbm.at[pl.ds(base + c * CHUNK, CHUNK)]

        @pl.when(sid == 0)
        def _():
            pltpu.sync_copy(table_hbm, tab_sh)

        plsc.subcore_barrier()

        pltpu.async_copy(idx_src(0), idx0, isem0)
        pltpu.async_copy(idx_src(1), idx1, isem1)
        pltpu.make_async_copy(idx_src(0), idx0, isem0).wait()
        pltpu.async_copy(tab_sh.at[idx0], rows0, gsem0)

        def body(g, carry):
            c0 = 2 * g
            more = g + 1 < pair_steps

            pltpu.make_async_copy(idx_src(c0 + 1), idx1, isem1).wait()
            pltpu.async_copy(tab_sh.at[idx1], rows1, gsem1)

            pltpu.make_async_copy(tab_sh.at[idx0], rows0, gsem0).wait()

            @pl.when(more)
            def _():
                pltpu.async_copy(idx_src(c0 + 2), idx0, isem0)

            pltpu.sync_copy(rows0, out_dst(c0))

            @pl.when(more)
            def _():
                pltpu.make_async_copy(idx_src(c0 + 2), idx0, isem0).wait()
                pltpu.async_copy(tab_sh.at[idx0], rows0, gsem0)

            pltpu.make_async_copy(tab_sh.at[idx1], rows1, gsem1).wait()

            @pl.when(more)
            def _():
                pltpu.async_copy(idx_src(c0 + 3), idx1, isem1)

            pltpu.sync_copy(rows1, out_dst(c0 + 1))
            return carry

        lax.fori_loop(0, pair_steps, body, 0)

    return lookup


def kernel(weights, positions):
    flat = positions.reshape(-1).astype(jnp.int32)
    out = _make_lookup(flat.shape[0])(weights, flat)
    return out.reshape(*positions.shape, EMBED_DIM)
